# Initial kernel scaffold; baseline (speedup 1.0000x reference)
import functools

import jax
import jax.numpy as jnp
from jax import lax
from jax.experimental import pallas as pl
from jax.experimental.pallas import tpu as pltpu
from jax.experimental.pallas import tpu_sc as plsc

N_USERS = 6000
N_ITEMS = 4000
N_NODES = N_USERS + N_ITEMS
N_EDGES = 320000
DIM = 128
N_LAYERS = 3

NC = 2
NS = 16
TILES = NC * NS
CH = 128
NCH = (N_EDGES + TILES * CH - 1) // (TILES * CH)
EPT = NCH * CH
E_PAD = TILES * EPT
N_PAD = 10240
RPT = N_PAD // TILES
RPS = N_PAD // NS

_mesh = plsc.VectorSubcoreMesh(core_axis_name="c", subcore_axis_name="s")


def _scale_rows(buf, val_ref, j, n_rows):
    def edge_body(e, _):
        v = val_ref[j, e]
        for q in range(DIM // 16):
            sl = pl.ds(q * 16, 16)
            buf[e, sl] = buf[e, sl] * v
        return _
    lax.fori_loop(0, n_rows, edge_body, None)


@functools.partial(
    pl.kernel,
    out_type=jax.ShapeDtypeStruct((NC, N_PAD, DIM), jnp.float32),
    mesh=_mesh,
    scratch_types=[
        pltpu.VMEM((NCH, CH), jnp.int32),
        pltpu.VMEM((NCH, CH), jnp.int32),
        pltpu.VMEM((NCH, CH), jnp.float32),
        pltpu.VMEM((CH, DIM), jnp.float32),
        pltpu.VMEM((CH, DIM), jnp.float32),
        pltpu.VMEM_SHARED((N_PAD, DIM), jnp.float32),
        pltpu.SemaphoreType.DMA,
    ],
)
def _spmm_layer(row_h, col_h, val_h, x_h, out_h,
                row_v, col_v, val_v, gbuf, zbuf, acc, sem):
    c = lax.axis_index("c")
    s = lax.axis_index("s")
    tid = c * NS + s

    def zrow(r, _):
        for q in range(DIM // 16):
            zbuf[r, pl.ds(q * 16, 16)] = jnp.zeros((16,), jnp.float32)
        return _
    lax.fori_loop(0, CH, zrow, None)
    for b in range(RPS // CH):
        pltpu.sync_copy(zbuf, acc.at[pl.ds(s * RPS + b * CH, CH)])

    pltpu.sync_copy(row_h.at[tid], row_v)
    pltpu.sync_copy(col_h.at[tid], col_v)
    pltpu.sync_copy(val_h.at[tid], val_v)
    plsc.subcore_barrier()

    def chunk(j, _):
        pltpu.async_copy(x_h.at[col_v.at[j]], gbuf, sem).wait()
        _scale_rows(gbuf, val_v, j, CH)
        pltpu.sync_copy(gbuf, acc.at[row_v.at[j]], add=True)
        return _
    lax.fori_loop(0, NCH, chunk, None)

    plsc.subcore_barrier()
    pltpu.sync_copy(acc.at[pl.ds(s * RPS, RPS)],
                    out_h.at[c, pl.ds(s * RPS, RPS)])


_CB = 64


@functools.partial(
    pl.kernel,
    out_type=jax.ShapeDtypeStruct((N_PAD, DIM), jnp.float32),
    mesh=_mesh,
    scratch_types=[
        pltpu.VMEM((_CB, DIM), jnp.float32),
        pltpu.VMEM((_CB, DIM), jnp.float32),
    ],
)
def _combine(p_h, x_h, a, b):
    c = lax.axis_index("c")
    s = lax.axis_index("s")
    tid = c * NS + s
    for t in range(RPT // _CB):
        start = tid * RPT + t * _CB
        pltpu.sync_copy(p_h.at[0, pl.ds(start, _CB)], a)
        pltpu.sync_copy(p_h.at[1, pl.ds(start, _CB)], b)

        def rbody(r, _):
            for q in range(DIM // 16):
                sl = pl.ds(q * 16, 16)
                a[r, sl] = a[r, sl] + b[r, sl]
            return _
        lax.fori_loop(0, _CB, rbody, None)
        pltpu.sync_copy(a, x_h.at[pl.ds(start, _CB)])


@functools.partial(
    pl.kernel,
    out_type=jax.ShapeDtypeStruct((N_PAD, DIM), jnp.float32),
    mesh=_mesh,
    scratch_types=[
        pltpu.VMEM((_CB, DIM), jnp.float32),
        pltpu.VMEM((_CB, DIM), jnp.float32),
        pltpu.VMEM((_CB, DIM), jnp.float32),
        pltpu.VMEM((_CB, DIM), jnp.float32),
        pltpu.VMEM((_CB, DIM), jnp.float32),
    ],
)
def _finalize(x0_h, x1_h, x2_h, p3_h, out_h, a, b, d, e, f):
    c = lax.axis_index("c")
    s = lax.axis_index("s")
    tid = c * NS + s
    for t in range(RPT // _CB):
        start = tid * RPT + t * _CB
        pltpu.sync_copy(x0_h.at[pl.ds(start, _CB)], a)
        pltpu.sync_copy(x1_h.at[pl.ds(start, _CB)], b)
        pltpu.sync_copy(x2_h.at[pl.ds(start, _CB)], d)
        pltpu.sync_copy(p3_h.at[0, pl.ds(start, _CB)], e)
        pltpu.sync_copy(p3_h.at[1, pl.ds(start, _CB)], f)

        def rbody(r, _):
            for q in range(DIM // 16):
                sl = pl.ds(q * 16, 16)
                tot = (((a[r, sl] + b[r, sl]) + (d[r, sl] + e[r, sl]))
                       + f[r, sl])
                a[r, sl] = tot * 0.25
            return _
        lax.fori_loop(0, _CB, rbody, None)
        pltpu.sync_copy(a, out_h.at[pl.ds(start, _CB)])


def kernel(adj_indices, adj_values, user_weight, item_weight):
    row = adj_indices[0].astype(jnp.int32)
    col = adj_indices[1].astype(jnp.int32)
    val = adj_values.astype(jnp.float32)
    pad = E_PAD - N_EDGES
    row_p = jnp.concatenate([row, jnp.zeros((pad,), jnp.int32)]) \
        .reshape(TILES, NCH, CH)
    col_p = jnp.concatenate([col, jnp.zeros((pad,), jnp.int32)]) \
        .reshape(TILES, NCH, CH)
    val_p = jnp.concatenate([val, jnp.zeros((pad,), jnp.float32)]) \
        .reshape(TILES, NCH, CH)

    x0 = jnp.zeros((N_PAD, DIM), jnp.float32)
    x0 = x0.at[:N_USERS].set(user_weight)
    x0 = x0.at[N_USERS:N_NODES].set(item_weight)

    p1 = _spmm_layer(row_p, col_p, val_p, x0)
    x1 = _combine(p1)
    p2 = _spmm_layer(row_p, col_p, val_p, x1)
    x2 = _combine(p2)
    p3 = _spmm_layer(row_p, col_p, val_p, x2)
    fin = _finalize(x0, x1, x2, p3)

    return (fin[:N_USERS], fin[N_USERS:N_NODES])


# speedup vs baseline: 3.9076x; 3.9076x over previous
import functools

import jax
import jax.numpy as jnp
from jax import lax
from jax.experimental import pallas as pl
from jax.experimental.pallas import tpu as pltpu
from jax.experimental.pallas import tpu_sc as plsc

N_USERS = 6000
N_ITEMS = 4000
N_NODES = N_USERS + N_ITEMS
N_EDGES = 320000
DIM = 128
N_LAYERS = 3

NC = 2
NS = 16
TILES = NC * NS
CH = 128
NCH = (N_EDGES + TILES * CH - 1) // (TILES * CH)
EPT = NCH * CH
E_PAD = TILES * EPT
N_PAD = 10240
RPT = N_PAD // TILES
RPS = N_PAD // NS

_mesh = plsc.VectorSubcoreMesh(core_axis_name="c", subcore_axis_name="s")


def _scale_rows(buf, val_ref, j, n_rows):
    def group_body(g, _):
        vals = val_ref[j, pl.ds(g * 16, 16)]
        for l in range(16):
            v = vals[l]
            e = g * 16 + l
            for q in range(DIM // 16):
                sl = pl.ds(q * 16, 16)
                buf[e, sl] = buf[e, sl] * v
        return _
    lax.fori_loop(0, n_rows // 16, group_body, None)


@functools.partial(
    pl.kernel,
    out_type=jax.ShapeDtypeStruct((NC, N_PAD, DIM), jnp.float32),
    mesh=_mesh,
    scratch_types=[
        pltpu.VMEM((NCH, CH), jnp.int32),
        pltpu.VMEM((NCH, CH), jnp.int32),
        pltpu.VMEM((NCH, CH), jnp.float32),
        pltpu.VMEM((CH, DIM), jnp.float32),
        pltpu.VMEM_SHARED((N_PAD, DIM), jnp.float32),
        pltpu.SemaphoreType.DMA,
    ],
)
def _spmm_layer(row_h, col_h, val_h, x_h, out_h,
                row_v, col_v, val_v, gbuf, acc, sem):
    c = lax.axis_index("c")
    s = lax.axis_index("s")
    tid = c * NS + s

    def zrow(r, _):
        for q in range(DIM // 16):
            gbuf[r, pl.ds(q * 16, 16)] = jnp.zeros((16,), jnp.float32)
        return _
    lax.fori_loop(0, CH, zrow, None)
    for b in range(RPS // CH):
        pltpu.sync_copy(gbuf, acc.at[pl.ds(s * RPS + b * CH, CH)])

    pltpu.sync_copy(row_h.at[tid], row_v)
    pltpu.sync_copy(col_h.at[tid], col_v)
    pltpu.sync_copy(val_h.at[tid], val_v)
    plsc.subcore_barrier()

    def chunk(j, _):
        pltpu.async_copy(x_h.at[col_v.at[j]], gbuf, sem).wait()
        _scale_rows(gbuf, val_v, j, CH)
        pltpu.sync_copy(gbuf, acc.at[row_v.at[j]], add=True)
        return _
    lax.fori_loop(0, NCH, chunk, None)

    plsc.subcore_barrier()
    pltpu.sync_copy(acc.at[pl.ds(s * RPS, RPS)],
                    out_h.at[c, pl.ds(s * RPS, RPS)])


_CB = 64


@functools.partial(
    pl.kernel,
    out_type=jax.ShapeDtypeStruct((N_PAD, DIM), jnp.float32),
    mesh=_mesh,
    scratch_types=[
        pltpu.VMEM((_CB, DIM), jnp.float32),
        pltpu.VMEM((_CB, DIM), jnp.float32),
    ],
)
def _combine(p_h, x_h, a, b):
    c = lax.axis_index("c")
    s = lax.axis_index("s")
    tid = c * NS + s
    for t in range(RPT // _CB):
        start = tid * RPT + t * _CB
        pltpu.sync_copy(p_h.at[0, pl.ds(start, _CB)], a)
        pltpu.sync_copy(p_h.at[1, pl.ds(start, _CB)], b)

        def rbody(r, _):
            for q in range(DIM // 16):
                sl = pl.ds(q * 16, 16)
                a[r, sl] = a[r, sl] + b[r, sl]
            return _
        lax.fori_loop(0, _CB, rbody, None)
        pltpu.sync_copy(a, x_h.at[pl.ds(start, _CB)])


@functools.partial(
    pl.kernel,
    out_type=jax.ShapeDtypeStruct((N_PAD, DIM), jnp.float32),
    mesh=_mesh,
    scratch_types=[
        pltpu.VMEM((_CB, DIM), jnp.float32),
        pltpu.VMEM((_CB, DIM), jnp.float32),
        pltpu.VMEM((_CB, DIM), jnp.float32),
        pltpu.VMEM((_CB, DIM), jnp.float32),
        pltpu.VMEM((_CB, DIM), jnp.float32),
    ],
)
def _finalize(x0_h, x1_h, x2_h, p3_h, out_h, a, b, d, e, f):
    c = lax.axis_index("c")
    s = lax.axis_index("s")
    tid = c * NS + s
    for t in range(RPT // _CB):
        start = tid * RPT + t * _CB
        pltpu.sync_copy(x0_h.at[pl.ds(start, _CB)], a)
        pltpu.sync_copy(x1_h.at[pl.ds(start, _CB)], b)
        pltpu.sync_copy(x2_h.at[pl.ds(start, _CB)], d)
        pltpu.sync_copy(p3_h.at[0, pl.ds(start, _CB)], e)
        pltpu.sync_copy(p3_h.at[1, pl.ds(start, _CB)], f)

        def rbody(r, _):
            for q in range(DIM // 16):
                sl = pl.ds(q * 16, 16)
                tot = (((a[r, sl] + b[r, sl]) + (d[r, sl] + e[r, sl]))
                       + f[r, sl])
                a[r, sl] = tot * 0.25
            return _
        lax.fori_loop(0, _CB, rbody, None)
        pltpu.sync_copy(a, out_h.at[pl.ds(start, _CB)])


def kernel(adj_indices, adj_values, user_weight, item_weight):
    row = adj_indices[0].astype(jnp.int32)
    col = adj_indices[1].astype(jnp.int32)
    val = adj_values.astype(jnp.float32)
    pad = E_PAD - N_EDGES
    row_p = jnp.concatenate([row, jnp.zeros((pad,), jnp.int32)]) \
        .reshape(TILES, NCH, CH)
    col_p = jnp.concatenate([col, jnp.zeros((pad,), jnp.int32)]) \
        .reshape(TILES, NCH, CH)
    val_p = jnp.concatenate([val, jnp.zeros((pad,), jnp.float32)]) \
        .reshape(TILES, NCH, CH)

    x0 = jnp.zeros((N_PAD, DIM), jnp.float32)
    x0 = x0.at[:N_USERS].set(user_weight)
    x0 = x0.at[N_USERS:N_NODES].set(item_weight)

    p1 = _spmm_layer(row_p, col_p, val_p, x0)
    x1 = _combine(p1)
    p2 = _spmm_layer(row_p, col_p, val_p, x1)
    x2 = _combine(p2)
    p3 = _spmm_layer(row_p, col_p, val_p, x2)
    fin = _finalize(x0, x1, x2, p3)

    return (fin[:N_USERS], fin[N_USERS:N_NODES])
